# Initial kernel scaffold; baseline (speedup 1.0000x reference)
import jax
import jax.numpy as jnp
from jax.experimental import pallas as pl


def kernel(x, path_src, path_dst, path_len, b):
    raise NotImplementedError("write your pallas kernel here")


# speedup vs baseline: 2.5652x; 2.5652x over previous
import functools

import jax
import jax.numpy as jnp
from jax import lax
from jax.experimental import pallas as pl
from jax.experimental.pallas import tpu as pltpu, tpu_sc as plsc

N = 4096
NP = 1_000_000
MAXPD = 20
NW = 32
NB = 256
ROWS_PB = N // NB
CHUNK = ROWS_PB * N
VREGS = NP // 16
QV = -(-VREGS // NW)
WIN = 64
NFULL = QV // WIN
TAIL = QV - NFULL * WIN
IDXB = 128
NPAD = NP + 8 * NB + WIN * 16 + 16
W3 = 2048

_mesh = plsc.VectorSubcoreMesh(core_axis_name="c", subcore_axis_name="s")
_params = pltpu.CompilerParams(needs_layout_passes=False)


def _wid():
    return lax.axis_index("c") * 16 + lax.axis_index("s")


def _splat(x):
    return jnp.full((16,), x, jnp.int32)


def _vext0(vec):
    lane = lax.iota(jnp.int32, 16)
    return jnp.sum(jnp.where(lane == 0, vec, jnp.zeros((16,), vec.dtype)))


def _sget(ref, idx_scalar):
    return _vext0(plsc.load_gather(ref, [_splat(idx_scalar)]))


@functools.partial(
    pl.kernel,
    out_type=jax.ShapeDtypeStruct((NW * NB,), jnp.int32),
    mesh=_mesh,
    compiler_params=_params,
    scratch_types=[
        pltpu.VMEM((WIN * 16,), jnp.int32),
        pltpu.VMEM((NB,), jnp.int32),
    ],
)
def _k1(src_hbm, hist_out, win_ref, hist_ref):
    w = _wid()
    base = w * QV
    ones = jnp.ones((16,), jnp.int32)
    for i in range(NB // 16):
        hist_ref[pl.ds(i * 16, 16)] = jnp.zeros((16,), jnp.int32)

    def fullwin(j, carry):
        pltpu.sync_copy(src_hbm.at[pl.ds((base + j * WIN) * 16, WIN * 16)],
                        win_ref)
        for v in range(WIN):
            srcv = win_ref[pl.ds(v * 16, 16)]
            plsc.addupdate_scatter(hist_ref, [srcv >> 4], ones)
        return carry

    lax.fori_loop(0, NFULL, fullwin, 0)

    tb = jnp.minimum(base + NFULL * WIN, VREGS - TAIL)
    vlo = base + NFULL * WIN - tb
    pltpu.sync_copy(src_hbm.at[pl.ds(tb * 16, TAIL * 16)],
                    win_ref.at[pl.ds(0, TAIL * 16)])
    for v in range(TAIL):
        m = _splat(v) >= _splat(vlo)
        srcv = win_ref[pl.ds(v * 16, 16)]
        plsc.addupdate_scatter(hist_ref, [srcv >> 4], ones, mask=m)

    pltpu.sync_copy(hist_ref, hist_out.at[pl.ds(w * NB, NB)])


@functools.partial(
    pl.kernel,
    out_type=[
        jax.ShapeDtypeStruct((NPAD,), jnp.int32),
        jax.ShapeDtypeStruct((NPAD,), jnp.float32),
    ],
    mesh=_mesh,
    compiler_params=_params,
    scratch_types=[
        pltpu.VMEM((NW * NB,), jnp.int32),
        pltpu.VMEM((NB,), jnp.int32),
        pltpu.VMEM((32,), jnp.float32),
        pltpu.VMEM((WIN * 16,), jnp.int32),
        pltpu.VMEM((WIN * 16,), jnp.int32),
        pltpu.VMEM((WIN * 16,), jnp.int32),
        pltpu.VMEM((WIN * 16 // IDXB, IDXB), jnp.int32),
        pltpu.VMEM((WIN * 16 // IDXB, IDXB), jnp.float32),
        pltpu.VMEM((WIN * 16 // IDXB, IDXB), jnp.int32),
        pltpu.SemaphoreType.DMA,
    ],
)
def _k2(src_hbm, dst_hbm, len_hbm, b_hbm, hist_hbm, cell_out, val_out,
        hist_ref, cur_ref, b_ref, swin, dwin, lwin, cstage, vstage, ostage,
        sem):
    w = _wid()
    wv = _splat(w)
    pltpu.sync_copy(hist_hbm, hist_ref)
    pltpu.sync_copy(b_hbm, b_ref.at[pl.ds(0, MAXPD)])

    carry = jnp.int32(0)
    for bv in range(NB // 16):
        tot = jnp.zeros((16,), jnp.int32)
        pre = jnp.zeros((16,), jnp.int32)
        for wi in range(NW):
            h = hist_ref[pl.ds(wi * NB + bv * 16, 16)]
            tot = tot + h
            pre = pre + jnp.where(_splat(wi) < wv, h, jnp.zeros((16,), jnp.int32))
        tota = jnp.bitwise_and(tot + 7, _splat(-8))
        cs = plsc.cumsum(tota)
        cur_ref[pl.ds(bv * 16, 16)] = cs - tota + _splat(carry) + pre
        carry = carry + jnp.sum(tota)

    base = w * QV

    def _emit_vreg(srcv, dstv, lenv, mask):
        bucket = srcv >> 4
        cellv = ((srcv & 15) << 12) | dstv
        bidx = jnp.minimum(jnp.maximum(lenv, 1), MAXPD) - 1
        valv = plsc.load_gather(b_ref, [bidx])
        if mask is None:
            cnt, _ = plsc.scan_count(bucket)
            cur = plsc.load_gather(cur_ref, [bucket])
            dest = cur + cnt - 1
            plsc.store_scatter(cur_ref, [bucket], cur + cnt)
        else:
            cnt, _ = plsc.scan_count(bucket, mask=mask)
            cur = plsc.load_gather(cur_ref, [bucket])
            dest = cur + cnt - 1
            plsc.store_scatter(cur_ref, [bucket], cur + cnt, mask=mask)
            pad = _splat(NPAD - 16) + lax.iota(jnp.int32, 16)
            dest = jnp.where(mask, dest, pad)
        return cellv, valv, dest

    def window(j, carry):
        off = (base + j * WIN) * 16
        pltpu.sync_copy(src_hbm.at[pl.ds(off, WIN * 16)], swin)
        pltpu.sync_copy(dst_hbm.at[pl.ds(off, WIN * 16)], dwin)
        pltpu.sync_copy(len_hbm.at[pl.ds(off, WIN * 16)], lwin)
        for v in range(WIN):
            cellv, valv, dest = _emit_vreg(
                swin[pl.ds(v * 16, 16)], dwin[pl.ds(v * 16, 16)],
                lwin[pl.ds(v * 16, 16)], None)
            r, q = (v * 16) // IDXB, (v * 16) % IDXB
            cstage[r, pl.ds(q, 16)] = cellv
            vstage[r, pl.ds(q, 16)] = valv
            ostage[r, pl.ds(q, 16)] = dest
        handles = []
        for r in range(WIN * 16 // IDXB):
            handles.append(pltpu.async_copy(
                cstage.at[r], cell_out.at[ostage.at[r]], sem))
            handles.append(pltpu.async_copy(
                vstage.at[r], val_out.at[ostage.at[r]], sem))
        for h in handles:
            h.wait()
        return carry

    lax.fori_loop(0, NFULL, window, 0)

    tb = jnp.minimum(base + NFULL * WIN, VREGS - TAIL)
    vlo = base + NFULL * WIN - tb
    pltpu.sync_copy(src_hbm.at[pl.ds(tb * 16, TAIL * 16)],
                    swin.at[pl.ds(0, TAIL * 16)])
    pltpu.sync_copy(dst_hbm.at[pl.ds(tb * 16, TAIL * 16)],
                    dwin.at[pl.ds(0, TAIL * 16)])
    pltpu.sync_copy(len_hbm.at[pl.ds(tb * 16, TAIL * 16)],
                    lwin.at[pl.ds(0, TAIL * 16)])
    for v in range(TAIL):
        m = _splat(v) >= _splat(vlo)
        cellv, valv, dest = _emit_vreg(
            swin[pl.ds(v * 16, 16)], dwin[pl.ds(v * 16, 16)],
            lwin[pl.ds(v * 16, 16)], m)
        r, q = (v * 16) // IDXB, (v * 16) % IDXB
        cstage[r, pl.ds(q, 16)] = cellv
        vstage[r, pl.ds(q, 16)] = valv
        ostage[r, pl.ds(q, 16)] = dest
    handles = []
    for r in range(-(-TAIL * 16 // IDXB)):
        handles.append(pltpu.async_copy(
            cstage.at[r], cell_out.at[ostage.at[r]], sem))
        handles.append(pltpu.async_copy(
            vstage.at[r], val_out.at[ostage.at[r]], sem))
    for h in handles:
        h.wait()


@functools.partial(
    pl.kernel,
    out_type=jax.ShapeDtypeStruct((N * N,), jnp.float32),
    mesh=_mesh,
    compiler_params=_params,
    scratch_types=[
        pltpu.VMEM((CHUNK,), jnp.float32),
        pltpu.VMEM((NW * NB,), jnp.int32),
        pltpu.VMEM((NB,), jnp.int32),
        pltpu.VMEM((NB,), jnp.int32),
        pltpu.VMEM((W3,), jnp.int32),
        pltpu.VMEM((W3,), jnp.float32),
    ],
)
def _k3(cell_hbm, val_hbm, hist_hbm, out_hbm,
        chunk, hist_ref, off_ref, tot_ref, cwin, vwin):
    w = _wid()
    pltpu.sync_copy(hist_hbm, hist_ref)

    carry = jnp.int32(0)
    for bv in range(NB // 16):
        tot = jnp.zeros((16,), jnp.int32)
        for wi in range(NW):
            tot = tot + hist_ref[pl.ds(wi * NB + bv * 16, 16)]
        tota = jnp.bitwise_and(tot + 7, _splat(-8))
        cs = plsc.cumsum(tota)
        off_ref[pl.ds(bv * 16, 16)] = cs - tota + _splat(carry)
        tot_ref[pl.ds(bv * 16, 16)] = tot
        carry = carry + jnp.sum(tota)

    def zbody(i, carry):
        for k in range(16):
            chunk[pl.ds(i * 256 + k * 16, 16)] = jnp.zeros((16,), jnp.float32)
        return carry

    lax.fori_loop(0, CHUNK // 256, zbody, 0)

    lane = lax.iota(jnp.int32, 16)
    zerof = jnp.zeros((16,), jnp.float32)

    def round_body(r, rcarry):
        b = w * (NB // NW) + r
        start = pl.multiple_of(_sget(off_ref, b), 8)
        tot = _sget(tot_ref, b)
        nwin = (tot + W3 - 1) // W3

        def swin_body(j, carry):
            rem = tot - j * W3
            pltpu.sync_copy(cell_hbm.at[pl.ds(start + j * W3, W3)], cwin)
            pltpu.sync_copy(val_hbm.at[pl.ds(start + j * W3, W3)], vwin)
            for v in range(W3 // 16):
                m = (_splat(v * 16) + lane) < _splat(rem)
                plsc.store_scatter(chunk, [cwin[pl.ds(v * 16, 16)]],
                                   vwin[pl.ds(v * 16, 16)], mask=m)
            return carry

        lax.fori_loop(0, nwin, swin_body, 0)

        pltpu.sync_copy(chunk,
                        out_hbm.at[pl.ds(pl.multiple_of(b * CHUNK, 8), CHUNK)])

        def zwin_body(j, carry):
            rem = tot - j * W3
            pltpu.sync_copy(cell_hbm.at[pl.ds(start + j * W3, W3)], cwin)
            for v in range(W3 // 16):
                m = (_splat(v * 16) + lane) < _splat(rem)
                plsc.store_scatter(chunk, [cwin[pl.ds(v * 16, 16)]],
                                   zerof, mask=m)
            return carry

        lax.fori_loop(0, nwin, zwin_body, 0)
        return rcarry

    lax.fori_loop(0, NB // NW, round_body, 0)


def kernel(x, path_src, path_dst, path_len, b):
    hist = _k1(path_src)
    cell, val = _k2(path_src, path_dst, path_len, b, hist)
    flat = _k3(cell, val, hist)
    return flat.reshape(N, N).astype(x.dtype)


# speedup vs baseline: 3.5002x; 1.3645x over previous
import functools

import jax
import jax.numpy as jnp
from jax import lax
from jax.experimental import pallas as pl
from jax.experimental.pallas import tpu as pltpu, tpu_sc as plsc

N = 4096
NP = 1_000_000
MAXPD = 20
NW = 32
NB = 256
ROWS_PB = N // NB
CHUNK = ROWS_PB * N
VREGS = NP // 16
QV = -(-VREGS // NW)
WIN = 64
NFULL = QV // WIN
TAIL = QV - NFULL * WIN
IDXB = 128
NPAD = NP + 8 * NB + WIN * 16 + 16
W3 = 2048

_mesh = plsc.VectorSubcoreMesh(core_axis_name="c", subcore_axis_name="s")
_params = pltpu.CompilerParams(needs_layout_passes=False)


def _wid():
    return lax.axis_index("c") * 16 + lax.axis_index("s")


def _splat(x):
    return jnp.full((16,), x, jnp.int32)


def _vext0(vec):
    lane = lax.iota(jnp.int32, 16)
    return jnp.sum(jnp.where(lane == 0, vec, jnp.zeros((16,), vec.dtype)))


def _sget(ref, idx_scalar):
    return _vext0(plsc.load_gather(ref, [_splat(idx_scalar)]))


@functools.partial(
    pl.kernel,
    out_type=jax.ShapeDtypeStruct((NW * NB,), jnp.int32),
    mesh=_mesh,
    compiler_params=_params,
    scratch_types=[
        pltpu.VMEM((WIN * 16,), jnp.int32),
        pltpu.VMEM((NB,), jnp.int32),
    ],
)
def _k1(src_hbm, hist_out, win_ref, hist_ref):
    w = _wid()
    base = w * QV
    ones = jnp.ones((16,), jnp.int32)
    for i in range(NB // 16):
        hist_ref[pl.ds(i * 16, 16)] = jnp.zeros((16,), jnp.int32)

    def fullwin(j, carry):
        pltpu.sync_copy(src_hbm.at[pl.ds((base + j * WIN) * 16, WIN * 16)],
                        win_ref)
        for v in range(WIN):
            srcv = win_ref[pl.ds(v * 16, 16)]
            plsc.addupdate_scatter(hist_ref, [srcv >> 4], ones)
        return carry

    lax.fori_loop(0, NFULL, fullwin, 0)

    tb = jnp.minimum(base + NFULL * WIN, VREGS - TAIL)
    vlo = base + NFULL * WIN - tb
    pltpu.sync_copy(src_hbm.at[pl.ds(tb * 16, TAIL * 16)],
                    win_ref.at[pl.ds(0, TAIL * 16)])
    for v in range(TAIL):
        m = _splat(v) >= _splat(vlo)
        srcv = win_ref[pl.ds(v * 16, 16)]
        plsc.addupdate_scatter(hist_ref, [srcv >> 4], ones, mask=m)

    pltpu.sync_copy(hist_ref, hist_out.at[pl.ds(w * NB, NB)])


@functools.partial(
    pl.kernel,
    out_type=jax.ShapeDtypeStruct((NPAD,), jnp.int32),
    mesh=_mesh,
    compiler_params=_params,
    scratch_types=[
        pltpu.VMEM((NW * NB,), jnp.int32),
        pltpu.VMEM((NB,), jnp.int32),
        pltpu.VMEM((WIN * 16,), jnp.int32),
        pltpu.VMEM((WIN * 16,), jnp.int32),
        pltpu.VMEM((WIN * 16,), jnp.int32),
        pltpu.VMEM((2, WIN * 16 // IDXB, IDXB), jnp.int32),
        pltpu.VMEM((2, WIN * 16 // IDXB, IDXB), jnp.int32),
        pltpu.SemaphoreType.DMA,
        pltpu.SemaphoreType.DMA,
    ],
)
def _k2(src_hbm, dst_hbm, len_hbm, hist_hbm, pair_out,
        hist_ref, cur_ref, swin, dwin, lwin, pstage, ostage, sem, osem):
    w = _wid()
    wv = _splat(w)
    pltpu.sync_copy(hist_hbm, hist_ref)

    carry = jnp.int32(0)
    for bv in range(NB // 16):
        tot = jnp.zeros((16,), jnp.int32)
        pre = jnp.zeros((16,), jnp.int32)
        for wi in range(NW):
            h = hist_ref[pl.ds(wi * NB + bv * 16, 16)]
            tot = tot + h
            pre = pre + jnp.where(_splat(wi) < wv, h, jnp.zeros((16,), jnp.int32))
        tota = jnp.bitwise_and(tot + 7, _splat(-8))
        cs = plsc.cumsum(tota)
        cur_ref[pl.ds(bv * 16, 16)] = cs - tota + _splat(carry) + pre
        carry = carry + jnp.sum(tota)

    base = w * QV
    nrows = WIN * 16 // IDXB

    def _emit_vreg(srcv, dstv, lenv, mask):
        bucket = srcv >> 4
        cellv = ((srcv & 15) << 12) | dstv
        bidx = jnp.minimum(jnp.maximum(lenv, 1), MAXPD) - 1
        packed = (cellv << 5) | bidx
        if mask is None:
            cnt, _ = plsc.scan_count(bucket)
            cur = plsc.load_gather(cur_ref, [bucket])
            dest = cur + cnt - 1
            plsc.store_scatter(cur_ref, [bucket], cur + cnt)
        else:
            cnt, _ = plsc.scan_count(bucket, mask=mask)
            cur = plsc.load_gather(cur_ref, [bucket])
            dest = cur + cnt - 1
            plsc.store_scatter(cur_ref, [bucket], cur + cnt, mask=mask)
            pad = _splat(NPAD - 16) + lax.iota(jnp.int32, 16)
            dest = jnp.where(mask, dest, pad)
        return packed, dest

    def _do_window(j, slot):
        off = (base + j * WIN) * 16
        h1 = pltpu.async_copy(src_hbm.at[pl.ds(off, WIN * 16)], swin, sem)
        h2 = pltpu.async_copy(dst_hbm.at[pl.ds(off, WIN * 16)], dwin, sem)
        h3 = pltpu.async_copy(len_hbm.at[pl.ds(off, WIN * 16)], lwin, sem)
        h1.wait(); h2.wait(); h3.wait()
        for v in range(WIN):
            packed, dest = _emit_vreg(
                swin[pl.ds(v * 16, 16)], dwin[pl.ds(v * 16, 16)],
                lwin[pl.ds(v * 16, 16)], None)
            r, q = (v * 16) // IDXB, (v * 16) % IDXB
            pstage[slot, r, pl.ds(q, 16)] = packed
            ostage[slot, r, pl.ds(q, 16)] = dest
        for r in range(nrows):
            pltpu.async_copy(pstage.at[slot, r],
                             pair_out.at[ostage.at[slot, r]], osem)

    def _drain(slot):
        for r in range(nrows):
            pltpu.make_async_copy(pstage.at[slot, r],
                                  pair_out.at[ostage.at[slot, r]], osem).wait()

    def pairwin(i, carry):
        @pl.when(i > 0)
        def _():
            _drain(0)
        _do_window(2 * i, 0)

        @pl.when(i > 0)
        def _():
            _drain(1)
        _do_window(2 * i + 1, 1)
        return carry

    lax.fori_loop(0, NFULL // 2, pairwin, 0)
    _drain(0)
    _drain(1)

    tb = jnp.minimum(base + NFULL * WIN, VREGS - TAIL)
    vlo = base + NFULL * WIN - tb
    pltpu.sync_copy(src_hbm.at[pl.ds(tb * 16, TAIL * 16)],
                    swin.at[pl.ds(0, TAIL * 16)])
    pltpu.sync_copy(dst_hbm.at[pl.ds(tb * 16, TAIL * 16)],
                    dwin.at[pl.ds(0, TAIL * 16)])
    pltpu.sync_copy(len_hbm.at[pl.ds(tb * 16, TAIL * 16)],
                    lwin.at[pl.ds(0, TAIL * 16)])
    for v in range(TAIL):
        m = _splat(v) >= _splat(vlo)
        packed, dest = _emit_vreg(
            swin[pl.ds(v * 16, 16)], dwin[pl.ds(v * 16, 16)],
            lwin[pl.ds(v * 16, 16)], m)
        r, q = (v * 16) // IDXB, (v * 16) % IDXB
        pstage[0, r, pl.ds(q, 16)] = packed
        ostage[0, r, pl.ds(q, 16)] = dest
    for v in range(TAIL, ((TAIL * 16 + IDXB - 1) // IDXB) * IDXB // 16):
        r, q = (v * 16) // IDXB, (v * 16) % IDXB
        ostage[0, r, pl.ds(q, 16)] = _splat(NPAD - 16) + lax.iota(jnp.int32, 16)
    handles = []
    for r in range(-(-TAIL * 16 // IDXB)):
        handles.append(pltpu.async_copy(
            pstage.at[0, r], pair_out.at[ostage.at[0, r]], osem))
    for h in handles:
        h.wait()


@functools.partial(
    pl.kernel,
    out_type=jax.ShapeDtypeStruct((N * N,), jnp.float32),
    mesh=_mesh,
    compiler_params=_params,
    scratch_types=[
        pltpu.VMEM((CHUNK,), jnp.float32),
        pltpu.VMEM((NW * NB,), jnp.int32),
        pltpu.VMEM((NB,), jnp.int32),
        pltpu.VMEM((NB,), jnp.int32),
        pltpu.VMEM((W3,), jnp.int32),
        pltpu.VMEM((32,), jnp.float32),
        pltpu.SemaphoreType.DMA,
    ],
)
def _k3(pair_hbm, hist_hbm, b_hbm, out_hbm,
        chunk, hist_ref, off_ref, tot_ref, pwin, b_ref, sem):
    w = _wid()
    pltpu.sync_copy(hist_hbm, hist_ref)
    pltpu.sync_copy(b_hbm, b_ref.at[pl.ds(0, MAXPD)])

    carry = jnp.int32(0)
    for bv in range(NB // 16):
        tot = jnp.zeros((16,), jnp.int32)
        for wi in range(NW):
            tot = tot + hist_ref[pl.ds(wi * NB + bv * 16, 16)]
        tota = jnp.bitwise_and(tot + 7, _splat(-8))
        cs = plsc.cumsum(tota)
        off_ref[pl.ds(bv * 16, 16)] = cs - tota + _splat(carry)
        tot_ref[pl.ds(bv * 16, 16)] = tot
        carry = carry + jnp.sum(tota)

    def zbody(i, carry):
        for k in range(16):
            chunk[pl.ds(i * 256 + k * 16, 16)] = jnp.zeros((16,), jnp.float32)
        return carry

    lax.fori_loop(0, CHUNK // 256, zbody, 0)

    lane = lax.iota(jnp.int32, 16)
    zerof = jnp.zeros((16,), jnp.float32)

    def round_body(r, rcarry):
        b = w * (NB // NW) + r
        start = pl.multiple_of(_sget(off_ref, b), 8)
        tot = _sget(tot_ref, b)
        nwin = (tot + W3 - 1) // W3

        def swin_body(j, carry):
            rem = tot - j * W3
            pltpu.sync_copy(pair_hbm.at[pl.ds(start + j * W3, W3)], pwin)
            for v in range(W3 // 16):
                m = (_splat(v * 16) + lane) < _splat(rem)
                p = pwin[pl.ds(v * 16, 16)]
                val = plsc.load_gather(b_ref, [p & 31])
                plsc.store_scatter(chunk, [p >> 5], val, mask=m)
            return carry

        lax.fori_loop(0, nwin, swin_body, 0)

        pltpu.sync_copy(chunk,
                        out_hbm.at[pl.ds(pl.multiple_of(b * CHUNK, 8), CHUNK)])

        def zwin_body(j, carry):
            rem = tot - j * W3
            pltpu.sync_copy(pair_hbm.at[pl.ds(start + j * W3, W3)], pwin)
            for v in range(W3 // 16):
                m = (_splat(v * 16) + lane) < _splat(rem)
                plsc.store_scatter(chunk, [pwin[pl.ds(v * 16, 16)] >> 5],
                                   zerof, mask=m)
            return carry

        lax.fori_loop(0, nwin, zwin_body, 0)
        return rcarry

    lax.fori_loop(0, NB // NW, round_body, 0)


def kernel(x, path_src, path_dst, path_len, b):
    hist = _k1(path_src)
    pair = _k2(path_src, path_dst, path_len, hist)
    flat = _k3(pair, hist, b)
    return flat.reshape(N, N).astype(x.dtype)
